# baseline (speedup 1.0000x reference)
import functools

import jax
import jax.numpy as jnp
from jax import lax
from jax.experimental import pallas as pl
from jax.experimental.pallas import tpu as pltpu
from jax.experimental.pallas import tpu_sc as plsc

_N_CORES = 2
_N_SUBCORES = 16
_LANES = 16
_NW = _N_CORES * _N_SUBCORES
_CHUNK = 256
_IPACK = 4


def _funk_svd_body(n_factors, b_per_w, uid_hbm, iid_hbm, uf_hbm, i2_hbm,
                   out_hbm, uid_v, iid_v, isup_v, u_rows, i_sup, out_v,
                   sem_u, sem_i):
    wid = lax.axis_index("s") * _N_CORES + lax.axis_index("c")
    base = wid * b_per_w

    pltpu.sync_copy(uid_hbm.at[pl.ds(base, b_per_w)], uid_v)
    pltpu.sync_copy(iid_hbm.at[pl.ds(base, b_per_w)], iid_v)

    def sup(g, carry):
        v = iid_v[pl.ds(g * _LANES, _LANES)]
        isup_v[pl.ds(g * _LANES, _LANES)] = lax.shift_right_logical(
            v, jnp.full((_LANES,), 2, jnp.int32))
        return carry

    lax.fori_loop(0, b_per_w // _LANES, sup, 0)

    lane = lax.iota(jnp.int32, _LANES)
    n_chunks = b_per_w // _CHUNK
    groups_per_chunk = _CHUNK // _LANES

    for c in range(n_chunks):
        item_cp = pltpu.async_copy(
            i2_hbm.at[isup_v.at[pl.ds(c * _CHUNK, _CHUNK)]],
            i_sup, sem_i)

        def fire(g, carry, c=c):
            uv = uid_v[pl.ds(c * _CHUNK + g * _LANES, _LANES)]
            for t in range(_LANES):
                pltpu.async_copy(uf_hbm.at[pl.ds(uv[t], 1), :],
                                 u_rows.at[pl.ds(g * _LANES + t, 1), :],
                                 sem_u)
            return carry

        lax.fori_loop(0, groups_per_chunk, fire, 0)
        pltpu.make_async_copy(uf_hbm.at[pl.ds(0, _CHUNK), :],
                              u_rows, sem_u).wait()
        item_cp.wait()

        def group(g, carry, c=c):
            rows = lane + g * _LANES
            iv = iid_v[pl.ds(c * _CHUNK + g * _LANES, _LANES)]
            colbase = lax.mul(
                lax.rem(iv, jnp.full((_LANES,), _IPACK, jnp.int32)),
                jnp.full((_LANES,), n_factors, jnp.int32))
            acc = jnp.zeros((_LANES,), jnp.float32)
            for f in range(n_factors):
                u = plsc.load_gather(u_rows, [rows, jnp.full((_LANES,), f, jnp.int32)])
                it = plsc.load_gather(i_sup, [rows, colbase + f])
                acc = acc + u * it
            out_v[pl.ds(c * _CHUNK + g * _LANES, _LANES)] = acc
            return carry

        lax.fori_loop(0, groups_per_chunk, group, 0)

    pltpu.sync_copy(out_v, out_hbm.at[pl.ds(base, b_per_w)])


def kernel(user_ids, item_ids, user_factors, item_factors):
    batch = user_ids.shape[0]
    n_items, n_factors = item_factors.shape
    b_per_w = batch // _NW
    item_packed = item_factors.reshape(n_items // _IPACK, _IPACK * n_factors)
    mesh = plsc.VectorSubcoreMesh(core_axis_name="c", subcore_axis_name="s")

    run = pl.kernel(
        functools.partial(_funk_svd_body, n_factors, b_per_w),
        out_type=jax.ShapeDtypeStruct((batch,), jnp.float32),
        mesh=mesh,
        compiler_params=pltpu.CompilerParams(needs_layout_passes=False),
        scratch_types=[
            pltpu.VMEM((b_per_w,), jnp.int32),
            pltpu.VMEM((b_per_w,), jnp.int32),
            pltpu.VMEM((b_per_w,), jnp.int32),
            pltpu.VMEM((_CHUNK, n_factors), jnp.float32),
            pltpu.VMEM((_CHUNK, _IPACK * n_factors), jnp.float32),
            pltpu.VMEM((b_per_w,), jnp.float32),
            pltpu.SemaphoreType.DMA,
            pltpu.SemaphoreType.DMA,
        ],
    )
    return run(user_ids.astype(jnp.int32), item_ids.astype(jnp.int32),
               user_factors, item_packed)
